# baseline (device time: 32996 ns/iter reference)
import jax
import jax.numpy as jnp
from jax import lax
from jax.experimental import pallas as pl
from jax.experimental.pallas import tpu as pltpu

N_DEV = 4


def kernel(ids, E):
    (T,) = ids.shape
    V, D = E.shape

    def body(ids_ref, e_ref, out_ref, comm_ref, send_sems, recv_sems):
        my_pos = lax.axis_index("i")
        left = (my_pos - 1) % N_DEV
        right = (my_pos + 1) % N_DEV

        barrier_sem = pltpu.get_barrier_semaphore()
        for nbr in [left, right]:
            pl.semaphore_signal(
                barrier_sem, inc=1,
                device_id=(nbr,), device_id_type=pl.DeviceIdType.MESH,
            )
        pl.semaphore_wait(barrier_sem, 2)

        local = ids_ref[:, :] - my_pos * V
        iota = lax.broadcasted_iota(jnp.int32, (T, V), 1)
        onehot = (local == iota).astype(jnp.bfloat16)
        partial = jnp.dot(
            onehot, e_ref[:, :].astype(jnp.bfloat16),
            preferred_element_type=jnp.float32,
        )
        out_ref[:, :] = partial
        comm_ref[0, :, :] = partial.astype(jnp.bfloat16)

        for h in range(N_DEV - 1):
            rdma = pltpu.make_async_remote_copy(
                src_ref=comm_ref.at[h],
                dst_ref=comm_ref.at[h + 1],
                send_sem=send_sems.at[h],
                recv_sem=recv_sems.at[h],
                device_id=(right,),
                device_id_type=pl.DeviceIdType.MESH,
            )
            rdma.start()
            rdma.wait()
            out_ref[:, :] += comm_ref[h + 1, :, :].astype(jnp.float32)

    return pl.pallas_call(
        body,
        out_shape=jax.ShapeDtypeStruct((T, D), jnp.float32),
        in_specs=[
            pl.BlockSpec(memory_space=pltpu.VMEM),
            pl.BlockSpec(memory_space=pltpu.VMEM),
        ],
        out_specs=pl.BlockSpec(memory_space=pltpu.VMEM),
        scratch_shapes=[
            pltpu.VMEM((N_DEV, T, D), jnp.bfloat16),
            pltpu.SemaphoreType.DMA((N_DEV - 1,)),
            pltpu.SemaphoreType.DMA((N_DEV - 1,)),
        ],
        compiler_params=pltpu.CompilerParams(collective_id=0),
    )(ids.reshape(T, 1), E)


# device time: 21582 ns/iter; 1.5289x vs baseline; 1.5289x over previous
import jax
import jax.numpy as jnp
from jax import lax
from jax.experimental import pallas as pl
from jax.experimental.pallas import tpu as pltpu

N_DEV = 4


def kernel(ids, E):
    (T,) = ids.shape
    V, D = E.shape
    H = T // 2
    Q = T // 4
    O = T // 8

    def body(ids_ref, e_ref, out_ref, p_ref, r1_ref, r2_ref,
             send_sems, recv_sems):
        p = lax.axis_index("i")
        pA = jnp.bitwise_xor(p, 1)
        pB = 3 - p

        barrier_sem = pltpu.get_barrier_semaphore()
        for nbr in [pA, pB]:
            pl.semaphore_signal(
                barrier_sem, inc=1,
                device_id=(nbr,), device_id_type=pl.DeviceIdType.MESH,
            )
        pl.semaphore_wait(barrier_sem, 2)

        local = ids_ref[:, :] - p * V
        iota = lax.broadcasted_iota(jnp.int32, (T, V), 1)
        onehot = (local == iota).astype(jnp.bfloat16)
        p_ref[:, :] = jnp.dot(
            onehot, e_ref[:, :].astype(jnp.bfloat16),
            preferred_element_type=jnp.float32,
        ).astype(jnp.bfloat16)

        k1 = jnp.where((p == 1) | (p == 2), 1, 0).astype(jnp.int32)
        k2 = p // 2
        m1 = p // 2
        m2 = jnp.bitwise_and(p, 1)

        def xchg(slot, src, dst, partner):
            return pltpu.make_async_remote_copy(
                src_ref=src, dst_ref=dst,
                send_sem=send_sems.at[slot], recv_sem=recv_sems.at[slot],
                device_id=(partner,), device_id_type=pl.DeviceIdType.MESH,
            )

        c10 = xchg(0, p_ref.at[pl.ds((1 - k1) * Q, Q), :], r1_ref.at[0], pA)
        c11 = xchg(1, p_ref.at[pl.ds(H + (1 - m1) * Q, Q), :], r1_ref.at[1], pB)
        c10.start()
        c11.start()
        c10.wait()
        c11.wait()
        ka = k1 * Q
        p_ref[pl.ds(ka, Q), :] = p_ref[pl.ds(ka, Q), :] + r1_ref[0, :, :]
        kb = H + m1 * Q
        p_ref[pl.ds(kb, Q), :] = p_ref[pl.ds(kb, Q), :] + r1_ref[1, :, :]

        c20 = xchg(2, p_ref.at[pl.ds(ka + (1 - k2) * O, O), :], r2_ref.at[0], pB)
        c21 = xchg(3, p_ref.at[pl.ds(kb + (1 - m2) * O, O), :], r2_ref.at[1], pA)
        c20.start()
        c21.start()
        c20.wait()
        c21.wait()
        o0 = ka + k2 * O
        p_ref[pl.ds(o0, O), :] = p_ref[pl.ds(o0, O), :] + r2_ref[0, :, :]
        o1 = kb + m2 * O
        p_ref[pl.ds(o1, O), :] = p_ref[pl.ds(o1, O), :] + r2_ref[1, :, :]

        c30 = xchg(4, p_ref.at[pl.ds(o0, O), :], p_ref.at[pl.ds(o0, O), :], pB)
        c31 = xchg(5, p_ref.at[pl.ds(o1, O), :], p_ref.at[pl.ds(o1, O), :], pA)
        c30.start()
        c31.start()
        c30.wait()
        c31.wait()

        c40 = xchg(6, p_ref.at[pl.ds(ka, Q), :], p_ref.at[pl.ds(ka, Q), :], pA)
        c41 = xchg(7, p_ref.at[pl.ds(kb, Q), :], p_ref.at[pl.ds(kb, Q), :], pB)
        c40.start()
        c41.start()
        c40.wait()
        c41.wait()

        out_ref[:, :] = p_ref[:, :].astype(jnp.float32)

    return pl.pallas_call(
        body,
        out_shape=jax.ShapeDtypeStruct((T, D), jnp.float32),
        in_specs=[
            pl.BlockSpec(memory_space=pltpu.VMEM),
            pl.BlockSpec(memory_space=pltpu.VMEM),
        ],
        out_specs=pl.BlockSpec(memory_space=pltpu.VMEM),
        scratch_shapes=[
            pltpu.VMEM((T, D), jnp.bfloat16),
            pltpu.VMEM((2, Q, D), jnp.bfloat16),
            pltpu.VMEM((2, O, D), jnp.bfloat16),
            pltpu.SemaphoreType.DMA((8,)),
            pltpu.SemaphoreType.DMA((8,)),
        ],
        compiler_params=pltpu.CompilerParams(collective_id=0),
    )(ids.reshape(T, 1), E)
